# baseline (speedup 1.0000x reference)
import functools

import jax
import jax.numpy as jnp
from jax import lax
from jax.experimental import pallas as pl
from jax.experimental.pallas import tpu as pltpu
from jax.experimental.pallas import tpu_sc as plsc

F32 = jnp.float32
BF16 = jnp.bfloat16
D = 128
BLK = 1024


def _round_up(v, m):
    return (v + m - 1) // m * m


def _row(i):
    return (i, 0)


def _cst(i):
    return (0, 0)


def _cst3(i):
    return (0, 0, 0)


def _full_specs(*arrs):
    return [pl.BlockSpec(a.shape, _cst3 if a.ndim == 3 else _cst) for a in arrs]


def _ln(xv, g, b):
    mu = jnp.mean(xv, axis=-1, keepdims=True)
    xc = xv - mu
    var = jnp.mean(xc * xc, axis=-1, keepdims=True)
    return xc * lax.rsqrt(var + 1e-5) * g + b


def _dot3(a, wp):
    ah = a.astype(BF16)
    al = (a - ah.astype(F32)).astype(BF16)
    return (
        jnp.dot(ah, wp[0], preferred_element_type=F32)
        + jnp.dot(al, wp[0], preferred_element_type=F32)
        + jnp.dot(ah, wp[1], preferred_element_type=F32)
    )


def _split_w(w):
    hi = w.astype(BF16)
    lo = (w - hi.astype(F32)).astype(BF16)
    return jnp.stack([hi, lo])


def _node_enc(x, w1, b1, w2, b2, w3, b3, g, b):
    n = x.shape[0]
    blk = min(BLK, n)

    def body(x_ref, w1r, b1r, w2r, b2r, w3r, b3r, gr, br, o_ref):
        v = jnp.maximum(_dot3(x_ref[...], w1r) + b1r[...], 0.0)
        v = jnp.maximum(_dot3(v, w2r) + b2r[...], 0.0)
        v = _dot3(v, w3r) + b3r[...]
        o_ref[...] = _ln(v, gr[...], br[...])

    return pl.pallas_call(
        body,
        grid=(n // blk,),
        in_specs=[pl.BlockSpec((blk, x.shape[1]), _row)] + _full_specs(w1, b1, w2, b2, w3, b3, g, b),
        out_specs=pl.BlockSpec((blk, D), _row),
        out_shape=jax.ShapeDtypeStruct((n, D), F32),
    )(x, w1, b1, w2, b2, w3, b3, g, b)


def _edge_enc(ea, w1, b1, w2, b2, w3, b3, g, b):
    n, din = ea.shape

    def body(ea_ref, w1r, b1r, w2r, b2r, w3r, b3r, gr, br, o_ref):
        acc = jnp.broadcast_to(b1r[...], (BLK, D))
        for k in range(din):
            acc = acc + ea_ref[:, k : k + 1] * w1r[k : k + 1, :]
        v = jnp.maximum(acc, 0.0)
        v = jnp.maximum(_dot3(v, w2r) + b2r[...], 0.0)
        v = _dot3(v, w3r) + b3r[...]
        o_ref[...] = _ln(v, gr[...], br[...])

    return pl.pallas_call(
        body,
        grid=(n // BLK,),
        in_specs=[pl.BlockSpec((BLK, din), _row)] + _full_specs(w1, b1, w2, b2, w3, b3, g, b),
        out_specs=pl.BlockSpec((BLK, D), _row),
        out_shape=jax.ShapeDtypeStruct((n, D), F32),
    )(ea, w1, b1, w2, b2, w3, b3, g, b)


def _uv(h, w1a, w1b):
    n = h.shape[0]
    blk = min(BLK, n)

    def body(h_ref, war, wbr, u_ref, v_ref):
        u_ref[...] = _dot3(h_ref[...], war)
        v_ref[...] = _dot3(h_ref[...], wbr)

    return pl.pallas_call(
        body,
        grid=(n // blk,),
        in_specs=[pl.BlockSpec((blk, D), _row)] + _full_specs(w1a, w1b),
        out_specs=[pl.BlockSpec((blk, D), _row)] * 2,
        out_shape=[jax.ShapeDtypeStruct((n, D), F32)] * 2,
    )(h, w1a, w1b)


def _edge_mlp(us, vd, e, b1, w1c, w2, b2, w3, b3, g, b):
    n = e.shape[0]

    def body(us_ref, vd_ref, e_ref, b1r, w1cr, w2r, b2r, w3r, b3r, gr, br, o_ref):
        v = us_ref[...] + vd_ref[...] + _dot3(e_ref[...], w1cr) + b1r[...]
        v = jnp.maximum(v, 0.0)
        v = jnp.maximum(_dot3(v, w2r) + b2r[...], 0.0)
        v = _dot3(v, w3r) + b3r[...]
        o_ref[...] = e_ref[...] + _ln(v, gr[...], br[...])

    return pl.pallas_call(
        body,
        grid=(n // BLK,),
        in_specs=[pl.BlockSpec((BLK, D), _row)] * 3 + _full_specs(b1, w1c, w2, b2, w3, b3, g, b),
        out_specs=pl.BlockSpec((BLK, D), _row),
        out_shape=jax.ShapeDtypeStruct((n, D), F32),
    )(us, vd, e, b1, w1c, w2, b2, w3, b3, g, b)


def _node_mlp(h, parts, w1h, w1a, b1, w2, b2, w3, b3, g, b):
    n = h.shape[0]
    blk = min(BLK, n)

    def body(h_ref, p_ref, w1hr, w1ar, b1r, w2r, b2r, w3r, b3r, gr, br, o_ref):
        agg = p_ref[0] + p_ref[1]
        v = _dot3(h_ref[...], w1hr) + _dot3(agg, w1ar) + b1r[...]
        v = jnp.maximum(v, 0.0)
        v = jnp.maximum(_dot3(v, w2r) + b2r[...], 0.0)
        v = _dot3(v, w3r) + b3r[...]
        o_ref[...] = h_ref[...] + _ln(v, gr[...], br[...])

    return pl.pallas_call(
        body,
        grid=(n // blk,),
        in_specs=[pl.BlockSpec((blk, D), _row), pl.BlockSpec((2, blk, D), lambda i: (0, i, 0))]
        + _full_specs(w1h, w1a, b1, w2, b2, w3, b3, g, b),
        out_specs=pl.BlockSpec((blk, D), _row),
        out_shape=jax.ShapeDtypeStruct((n, D), F32),
    )(h, parts, w1h, w1a, b1, w2, b2, w3, b3, g, b)


def _decoder(h, w1, b1, w2, b2, w3, b3):
    n = h.shape[0]
    blk = min(BLK, n)

    def body(h_ref, w1r, b1r, w2r, b2r, w3r, b3r, o_ref):
        v = jnp.maximum(_dot3(h_ref[...], w1r) + b1r[...], 0.0)
        v = jnp.maximum(_dot3(v, w2r) + b2r[...], 0.0)
        o_ref[...] = _dot3(v, w3r) + b3r[...]

    return pl.pallas_call(
        body,
        grid=(n // blk,),
        in_specs=[pl.BlockSpec((blk, D), _row)] + _full_specs(w1, b1, w2, b2, w3, b3),
        out_specs=pl.BlockSpec((blk, D), _row),
        out_shape=jax.ShapeDtypeStruct((n, D), F32),
    )(h, w1, b1, w2, b2, w3, b3)


def _sc_gather(u, v, src2d, dst2d):
    nb = src2d.shape[0]
    ep = nb * 128
    np_ = u.shape[0]
    rows = np_ // 16
    mesh = plsc.VectorSubcoreMesh(core_axis_name="c", subcore_axis_name="s")

    @functools.partial(
        pl.kernel,
        mesh=mesh,
        out_type=[
            jax.ShapeDtypeStruct((ep, D), F32),
            jax.ShapeDtypeStruct((ep, D), F32),
        ],
        scratch_types=[pltpu.VMEM_SHARED((np_, D), F32)],
    )
    def gk(u_hbm, v_hbm, s_hbm, d_hbm, us_hbm, vd_hbm, table_sh):
        cid = lax.axis_index("c")
        sid = lax.axis_index("s")
        sl = pl.ds(sid * rows, rows)

        @pl.when(cid == 0)
        def _():
            pltpu.sync_copy(u_hbm.at[sl], table_sh.at[sl])

        @pl.when(cid == 1)
        def _():
            pltpu.sync_copy(v_hbm.at[sl], table_sh.at[sl])

        plsc.subcore_barrier()

        def body(i_vmem, o_vmem):
            pltpu.sync_copy(table_sh.at[i_vmem.at[0]], o_vmem)

        pipe = functools.partial(
            pltpu.emit_pipeline,
            body,
            grid=(nb,),
            in_specs=[pl.BlockSpec((1, 128), _row)],
            out_specs=[pl.BlockSpec((128, D), _row)],
            core_axis_name="s",
            dimension_semantics=(pltpu.PARALLEL,),
        )

        @pl.when(cid == 0)
        def _():
            pipe()(s_hbm, us_hbm)

        @pl.when(cid == 1)
        def _():
            pipe()(d_hbm, vd_hbm)

    return gk(u, v, src2d, dst2d)


def _sc_gather_hbm(u, v, src2d, dst2d):
    nb = src2d.shape[0]
    ep = nb * 128
    mesh = plsc.VectorSubcoreMesh(core_axis_name="c", subcore_axis_name="s")

    @functools.partial(
        pl.kernel,
        mesh=mesh,
        out_type=[
            jax.ShapeDtypeStruct((ep, D), F32),
            jax.ShapeDtypeStruct((ep, D), F32),
        ],
    )
    def gk(u_hbm, v_hbm, s_hbm, d_hbm, us_hbm, vd_hbm):
        def body(s_vmem, d_vmem, us_vmem, vd_vmem):
            pltpu.sync_copy(u_hbm.at[s_vmem.at[0]], us_vmem)
            pltpu.sync_copy(v_hbm.at[d_vmem.at[0]], vd_vmem)

        pltpu.emit_pipeline(
            body,
            grid=(nb,),
            in_specs=[
                pl.BlockSpec((1, 128), _row),
                pl.BlockSpec((1, 128), _row),
            ],
            out_specs=[
                pl.BlockSpec((128, D), _row),
                pl.BlockSpec((128, D), _row),
            ],
            core_axis_name=("c", "s"),
            dimension_semantics=(pltpu.PARALLEL,),
        )(s_hbm, d_hbm, us_hbm, vd_hbm)

    return gk(u, v, src2d, dst2d)


def _sc_scatter(e_new, dst2d, zeros_blk):
    nb = dst2d.shape[0]
    rows = zeros_blk.shape[0]
    np_ = rows * 16
    mesh = plsc.VectorSubcoreMesh(core_axis_name="c", subcore_axis_name="s")

    @functools.partial(
        pl.kernel,
        mesh=mesh,
        out_type=jax.ShapeDtypeStruct((2, np_, D), F32),
        scratch_types=[pltpu.VMEM_SHARED((np_, D), F32)],
    )
    def sk(e_hbm, d_hbm, z_hbm, out_hbm, acc_shared):
        cid = lax.axis_index("c")
        sid = lax.axis_index("s")
        pltpu.sync_copy(z_hbm, acc_shared.at[pl.ds(sid * rows, rows)])
        plsc.subcore_barrier()

        def body(e_vmem, d_vmem):
            pltpu.sync_copy(e_vmem, acc_shared.at[d_vmem.at[0]], add=True)

        pltpu.emit_pipeline(
            body,
            grid=(nb,),
            in_specs=[
                pl.BlockSpec((128, D), _row),
                pl.BlockSpec((1, 128), _row),
            ],
            out_specs=[],
            core_axis_name=("c", "s"),
            dimension_semantics=(pltpu.PARALLEL,),
        )(e_hbm, d_hbm)

        plsc.subcore_barrier()
        pltpu.sync_copy(
            acc_shared.at[pl.ds(sid * rows, rows)],
            out_hbm.at[cid].at[pl.ds(sid * rows, rows)],
        )

    return sk(e_new, dst2d, zeros_blk)


def _sc_scatter2(e0, e1, d0, d1, zeros_blk):
    nb = d0.shape[0]
    rows = zeros_blk.shape[0]
    np_ = rows * 16
    mesh = plsc.VectorSubcoreMesh(core_axis_name="c", subcore_axis_name="s")

    @functools.partial(
        pl.kernel,
        mesh=mesh,
        out_type=jax.ShapeDtypeStruct((2, np_, D), F32),
        scratch_types=[pltpu.VMEM_SHARED((np_, D), F32)],
    )
    def sk(e0_hbm, e1_hbm, d0_hbm, d1_hbm, z_hbm, out_hbm, acc_shared):
        cid = lax.axis_index("c")
        sid = lax.axis_index("s")
        pltpu.sync_copy(z_hbm, acc_shared.at[pl.ds(sid * rows, rows)])
        plsc.subcore_barrier()

        def body(e_vmem, d_vmem):
            pltpu.sync_copy(e_vmem, acc_shared.at[d_vmem.at[0]], add=True)

        for e_hbm, d_hbm in ((e0_hbm, d0_hbm), (e1_hbm, d1_hbm)):
            pltpu.emit_pipeline(
                body,
                grid=(nb,),
                in_specs=[
                    pl.BlockSpec((128, D), _row),
                    pl.BlockSpec((1, 128), _row),
                ],
                out_specs=[],
                core_axis_name=("c", "s"),
                dimension_semantics=(pltpu.PARALLEL,),
            )(e_hbm, d_hbm)

        plsc.subcore_barrier()
        pltpu.sync_copy(
            acc_shared.at[pl.ds(sid * rows, rows)],
            out_hbm.at[cid].at[pl.ds(sid * rows, rows)],
        )

    return sk(e0, e1, d0, d1, zeros_blk)


def kernel(x, edge_index, edge_attr, params):
    n = x.shape[0]
    ne = edge_attr.shape[0]
    np_ = _round_up(n, 2048)
    ep = _round_up(ne, 8192)
    half = ep // 2
    hb = half // 128

    src = edge_index[0].astype(jnp.int32)
    dst = edge_index[1].astype(jnp.int32)
    pad_ids = (jnp.arange(ep - ne, dtype=jnp.int32) % (np_ - n)) + n
    src2d = jnp.concatenate([src, pad_ids]).reshape(ep // 128, 128)
    dst2d = jnp.concatenate([dst, pad_ids]).reshape(ep // 128, 128)
    schunks = (src2d[:hb], src2d[hb:])
    dchunks = (dst2d[:hb], dst2d[hb:])

    x_pad = jnp.pad(x, ((0, np_ - n), (0, 0)))
    ea_pad = jnp.pad(edge_attr, ((0, ep - ne), (0, 0)))
    ea_chunks = (ea_pad[:half], ea_pad[half:])
    zeros_blk = jnp.zeros((np_ // 16, D), F32)

    def unpack(p, split_first=True):
        lin = p["lin"]
        out = []
        for i, l in enumerate(lin):
            out.append(_split_w(l["w"]) if (split_first or i > 0) else l["w"])
            out.append(l["b"].reshape(1, -1))
        if "ln" in p:
            out.append(p["ln"]["g"].reshape(1, -1))
            out.append(p["ln"]["b"].reshape(1, -1))
        return out

    h = _node_enc(x_pad, *unpack(params["node_enc"]))
    enc_w = unpack(params["edge_enc"], split_first=False)
    e = [_edge_enc(ea_c, *enc_w) for ea_c in ea_chunks]

    for blk in params["blocks"]:
        w1 = blk["edge_mlp"]["lin"][0]["w"]
        ew = unpack(blk["edge_mlp"])[1:]
        w1cp = _split_w(w1[2 * D :])
        u, v = _uv(h, _split_w(w1[:D]), _split_w(w1[D : 2 * D]))
        gath = [_sc_gather_hbm(u, v, schunks[c], dchunks[c]) for c in range(2)]
        e = [
            _edge_mlp(gath[c][0], gath[c][1], e[c], ew[0], w1cp, *ew[1:])
            for c in range(2)
        ]
        parts = _sc_scatter2(e[0], e[1], dchunks[0], dchunks[1], zeros_blk)
        w1n = blk["node_mlp"]["lin"][0]["w"]
        nw = unpack(blk["node_mlp"])[1:]
        h = _node_mlp(h, parts, _split_w(w1n[:D]), _split_w(w1n[D:]), *nw)

    dw = unpack(params["node_dec"])
    w3 = params["node_dec"]["lin"][2]["w"]
    out_dim = w3.shape[1]
    w3p = _split_w(jnp.pad(w3, ((0, 0), (0, D - out_dim))))
    b3p = jnp.pad(dw[5], ((0, 0), (0, D - out_dim)))
    out = _decoder(h, dw[0], dw[1], dw[2], dw[3], w3p, b3p)
    return out[:n, :out_dim]


# speedup vs baseline: 1.1657x; 1.1657x over previous
import functools

import jax
import jax.numpy as jnp
from jax import lax
from jax.experimental import pallas as pl
from jax.experimental.pallas import tpu as pltpu
from jax.experimental.pallas import tpu_sc as plsc

F32 = jnp.float32
BF16 = jnp.bfloat16
D = 128
BLK = 1024


def _round_up(v, m):
    return (v + m - 1) // m * m


def _row(i):
    return (i, 0)


def _cst(i):
    return (0, 0)


def _cst3(i):
    return (0, 0, 0)


def _full_specs(*arrs):
    return [pl.BlockSpec(a.shape, _cst3 if a.ndim == 3 else _cst) for a in arrs]


def _ln(xv, g, b):
    mu = jnp.mean(xv, axis=-1, keepdims=True)
    xc = xv - mu
    var = jnp.mean(xc * xc, axis=-1, keepdims=True)
    return xc * lax.rsqrt(var + 1e-5) * g + b


def _dot3(a, wp):
    ah = a.astype(BF16)
    al = (a - ah.astype(F32)).astype(BF16)
    a3 = jnp.concatenate([ah, al, ah], axis=1)
    return jnp.dot(a3, wp[...], preferred_element_type=F32)


def _split_w(w):
    hi = w.astype(BF16)
    lo = (w - hi.astype(F32)).astype(BF16)
    return jnp.concatenate([hi, hi, lo], axis=0)


def _node_enc(x, w1, b1, w2, b2, w3, b3, g, b):
    n = x.shape[0]
    blk = min(BLK, n)

    def body(x_ref, w1r, b1r, w2r, b2r, w3r, b3r, gr, br, o_ref):
        v = jnp.maximum(_dot3(x_ref[...], w1r) + b1r[...], 0.0)
        v = jnp.maximum(_dot3(v, w2r) + b2r[...], 0.0)
        v = _dot3(v, w3r) + b3r[...]
        o_ref[...] = _ln(v, gr[...], br[...])

    return pl.pallas_call(
        body,
        grid=(n // blk,),
        in_specs=[pl.BlockSpec((blk, x.shape[1]), _row)] + _full_specs(w1, b1, w2, b2, w3, b3, g, b),
        out_specs=pl.BlockSpec((blk, D), _row),
        out_shape=jax.ShapeDtypeStruct((n, D), F32),
    )(x, w1, b1, w2, b2, w3, b3, g, b)


def _edge_enc(ea, w1, b1, w2, b2, w3, b3, g, b):
    n, din = ea.shape

    def body(ea_ref, w1r, b1r, w2r, b2r, w3r, b3r, gr, br, o_ref):
        acc = jnp.broadcast_to(b1r[...], (BLK, D))
        for k in range(din):
            acc = acc + ea_ref[:, k : k + 1] * w1r[k : k + 1, :]
        v = jnp.maximum(acc, 0.0)
        v = jnp.maximum(_dot3(v, w2r) + b2r[...], 0.0)
        v = _dot3(v, w3r) + b3r[...]
        o_ref[...] = _ln(v, gr[...], br[...])

    return pl.pallas_call(
        body,
        grid=(n // BLK,),
        in_specs=[pl.BlockSpec((BLK, din), _row)] + _full_specs(w1, b1, w2, b2, w3, b3, g, b),
        out_specs=pl.BlockSpec((BLK, D), _row),
        out_shape=jax.ShapeDtypeStruct((n, D), F32),
    )(ea, w1, b1, w2, b2, w3, b3, g, b)


def _uv(h, w1a, w1b):
    n = h.shape[0]
    blk = min(BLK, n)

    def body(h_ref, war, wbr, u_ref, v_ref):
        u_ref[...] = _dot3(h_ref[...], war)
        v_ref[...] = _dot3(h_ref[...], wbr)

    return pl.pallas_call(
        body,
        grid=(n // blk,),
        in_specs=[pl.BlockSpec((blk, D), _row)] + _full_specs(w1a, w1b),
        out_specs=[pl.BlockSpec((blk, D), _row)] * 2,
        out_shape=[jax.ShapeDtypeStruct((n, D), F32)] * 2,
    )(h, w1a, w1b)


def _edge_mlp(us, vd, e, b1, w1c, w2, b2, w3, b3, g, b):
    n = e.shape[0]

    def body(us_ref, vd_ref, e_ref, b1r, w1cr, w2r, b2r, w3r, b3r, gr, br, o_ref):
        v = us_ref[...] + vd_ref[...] + _dot3(e_ref[...], w1cr) + b1r[...]
        v = jnp.maximum(v, 0.0)
        v = jnp.maximum(_dot3(v, w2r) + b2r[...], 0.0)
        v = _dot3(v, w3r) + b3r[...]
        o_ref[...] = e_ref[...] + _ln(v, gr[...], br[...])

    return pl.pallas_call(
        body,
        grid=(n // BLK,),
        in_specs=[pl.BlockSpec((BLK, D), _row)] * 3 + _full_specs(b1, w1c, w2, b2, w3, b3, g, b),
        out_specs=pl.BlockSpec((BLK, D), _row),
        out_shape=jax.ShapeDtypeStruct((n, D), F32),
    )(us, vd, e, b1, w1c, w2, b2, w3, b3, g, b)


def _node_mlp(h, parts, w1h, w1a, b1, w2, b2, w3, b3, g, b):
    n = h.shape[0]
    blk = min(BLK, n)

    def body(h_ref, p_ref, w1hr, w1ar, b1r, w2r, b2r, w3r, b3r, gr, br, o_ref):
        agg = p_ref[0] + p_ref[1]
        v = _dot3(h_ref[...], w1hr) + _dot3(agg, w1ar) + b1r[...]
        v = jnp.maximum(v, 0.0)
        v = jnp.maximum(_dot3(v, w2r) + b2r[...], 0.0)
        v = _dot3(v, w3r) + b3r[...]
        o_ref[...] = h_ref[...] + _ln(v, gr[...], br[...])

    return pl.pallas_call(
        body,
        grid=(n // blk,),
        in_specs=[pl.BlockSpec((blk, D), _row), pl.BlockSpec((2, blk, D), lambda i: (0, i, 0))]
        + _full_specs(w1h, w1a, b1, w2, b2, w3, b3, g, b),
        out_specs=pl.BlockSpec((blk, D), _row),
        out_shape=jax.ShapeDtypeStruct((n, D), F32),
    )(h, parts, w1h, w1a, b1, w2, b2, w3, b3, g, b)


def _decoder(h, w1, b1, w2, b2, w3, b3):
    n = h.shape[0]
    blk = min(BLK, n)

    def body(h_ref, w1r, b1r, w2r, b2r, w3r, b3r, o_ref):
        v = jnp.maximum(_dot3(h_ref[...], w1r) + b1r[...], 0.0)
        v = jnp.maximum(_dot3(v, w2r) + b2r[...], 0.0)
        o_ref[...] = _dot3(v, w3r) + b3r[...]

    return pl.pallas_call(
        body,
        grid=(n // blk,),
        in_specs=[pl.BlockSpec((blk, D), _row)] + _full_specs(w1, b1, w2, b2, w3, b3),
        out_specs=pl.BlockSpec((blk, D), _row),
        out_shape=jax.ShapeDtypeStruct((n, D), F32),
    )(h, w1, b1, w2, b2, w3, b3)


def _sc_gather(u, v, src2d, dst2d):
    nb = src2d.shape[0]
    ep = nb * 128
    np_ = u.shape[0]
    rows = np_ // 16
    mesh = plsc.VectorSubcoreMesh(core_axis_name="c", subcore_axis_name="s")

    @functools.partial(
        pl.kernel,
        mesh=mesh,
        out_type=[
            jax.ShapeDtypeStruct((ep, D), F32),
            jax.ShapeDtypeStruct((ep, D), F32),
        ],
        scratch_types=[pltpu.VMEM_SHARED((np_, D), F32)],
    )
    def gk(u_hbm, v_hbm, s_hbm, d_hbm, us_hbm, vd_hbm, table_sh):
        cid = lax.axis_index("c")
        sid = lax.axis_index("s")
        sl = pl.ds(sid * rows, rows)

        @pl.when(cid == 0)
        def _():
            pltpu.sync_copy(u_hbm.at[sl], table_sh.at[sl])

        @pl.when(cid == 1)
        def _():
            pltpu.sync_copy(v_hbm.at[sl], table_sh.at[sl])

        plsc.subcore_barrier()

        def body(i_vmem, o_vmem):
            pltpu.sync_copy(table_sh.at[i_vmem.at[0]], o_vmem)

        pipe = functools.partial(
            pltpu.emit_pipeline,
            body,
            grid=(nb,),
            in_specs=[pl.BlockSpec((1, 128), _row)],
            out_specs=[pl.BlockSpec((128, D), _row)],
            core_axis_name="s",
            dimension_semantics=(pltpu.PARALLEL,),
        )

        @pl.when(cid == 0)
        def _():
            pipe()(s_hbm, us_hbm)

        @pl.when(cid == 1)
        def _():
            pipe()(d_hbm, vd_hbm)

    return gk(u, v, src2d, dst2d)


def _sc_gather_hbm(u, v, src2d, dst2d):
    nb = src2d.shape[0]
    ep = nb * 128
    mesh = plsc.VectorSubcoreMesh(core_axis_name="c", subcore_axis_name="s")

    @functools.partial(
        pl.kernel,
        mesh=mesh,
        out_type=[
            jax.ShapeDtypeStruct((ep, D), F32),
            jax.ShapeDtypeStruct((ep, D), F32),
        ],
    )
    def gk(u_hbm, v_hbm, s_hbm, d_hbm, us_hbm, vd_hbm):
        def body(s_vmem, d_vmem, us_vmem, vd_vmem):
            pltpu.sync_copy(u_hbm.at[s_vmem.at[0]], us_vmem)
            pltpu.sync_copy(v_hbm.at[d_vmem.at[0]], vd_vmem)

        pltpu.emit_pipeline(
            body,
            grid=(nb,),
            in_specs=[
                pl.BlockSpec((1, 128), _row),
                pl.BlockSpec((1, 128), _row),
            ],
            out_specs=[
                pl.BlockSpec((128, D), _row),
                pl.BlockSpec((128, D), _row),
            ],
            core_axis_name=("c", "s"),
            dimension_semantics=(pltpu.PARALLEL,),
        )(s_hbm, d_hbm, us_hbm, vd_hbm)

    return gk(u, v, src2d, dst2d)


def _sc_scatter(e_new, dst2d, zeros_blk):
    nb = dst2d.shape[0]
    rows = zeros_blk.shape[0]
    np_ = rows * 16
    mesh = plsc.VectorSubcoreMesh(core_axis_name="c", subcore_axis_name="s")

    @functools.partial(
        pl.kernel,
        mesh=mesh,
        out_type=jax.ShapeDtypeStruct((2, np_, D), F32),
        scratch_types=[pltpu.VMEM_SHARED((np_, D), F32)],
    )
    def sk(e_hbm, d_hbm, z_hbm, out_hbm, acc_shared):
        cid = lax.axis_index("c")
        sid = lax.axis_index("s")
        pltpu.sync_copy(z_hbm, acc_shared.at[pl.ds(sid * rows, rows)])
        plsc.subcore_barrier()

        def body(e_vmem, d_vmem):
            pltpu.sync_copy(e_vmem, acc_shared.at[d_vmem.at[0]], add=True)

        pltpu.emit_pipeline(
            body,
            grid=(nb,),
            in_specs=[
                pl.BlockSpec((128, D), _row),
                pl.BlockSpec((1, 128), _row),
            ],
            out_specs=[],
            core_axis_name=("c", "s"),
            dimension_semantics=(pltpu.PARALLEL,),
        )(e_hbm, d_hbm)

        plsc.subcore_barrier()
        pltpu.sync_copy(
            acc_shared.at[pl.ds(sid * rows, rows)],
            out_hbm.at[cid].at[pl.ds(sid * rows, rows)],
        )

    return sk(e_new, dst2d, zeros_blk)


def _sc_scatter2(e0, e1, d0, d1, zeros_blk):
    nb = d0.shape[0]
    rows = zeros_blk.shape[0]
    np_ = rows * 16
    mesh = plsc.VectorSubcoreMesh(core_axis_name="c", subcore_axis_name="s")

    @functools.partial(
        pl.kernel,
        mesh=mesh,
        out_type=jax.ShapeDtypeStruct((2, np_, D), F32),
        scratch_types=[pltpu.VMEM_SHARED((np_, D), F32)],
    )
    def sk(e0_hbm, e1_hbm, d0_hbm, d1_hbm, z_hbm, out_hbm, acc_shared):
        cid = lax.axis_index("c")
        sid = lax.axis_index("s")
        pltpu.sync_copy(z_hbm, acc_shared.at[pl.ds(sid * rows, rows)])
        plsc.subcore_barrier()

        def body(e_vmem, d_vmem):
            pltpu.sync_copy(e_vmem, acc_shared.at[d_vmem.at[0]], add=True)

        for e_hbm, d_hbm in ((e0_hbm, d0_hbm), (e1_hbm, d1_hbm)):
            pltpu.emit_pipeline(
                body,
                grid=(nb,),
                in_specs=[
                    pl.BlockSpec((128, D), _row),
                    pl.BlockSpec((1, 128), _row),
                ],
                out_specs=[],
                core_axis_name=("c", "s"),
                dimension_semantics=(pltpu.PARALLEL,),
            )(e_hbm, d_hbm)

        plsc.subcore_barrier()
        pltpu.sync_copy(
            acc_shared.at[pl.ds(sid * rows, rows)],
            out_hbm.at[cid].at[pl.ds(sid * rows, rows)],
        )

    return sk(e0, e1, d0, d1, zeros_blk)


def kernel(x, edge_index, edge_attr, params):
    n = x.shape[0]
    ne = edge_attr.shape[0]
    np_ = _round_up(n, 2048)
    ep = _round_up(ne, 8192)
    half = ep // 2
    hb = half // 128

    src = edge_index[0].astype(jnp.int32)
    dst = edge_index[1].astype(jnp.int32)
    pad_ids = (jnp.arange(ep - ne, dtype=jnp.int32) % (np_ - n)) + n
    src2d = jnp.concatenate([src, pad_ids]).reshape(ep // 128, 128)
    dst2d = jnp.concatenate([dst, pad_ids]).reshape(ep // 128, 128)
    schunks = (src2d[:hb], src2d[hb:])
    dchunks = (dst2d[:hb], dst2d[hb:])

    x_pad = jnp.pad(x, ((0, np_ - n), (0, 0)))
    ea_pad = jnp.pad(edge_attr, ((0, ep - ne), (0, 0)))
    ea_chunks = (ea_pad[:half], ea_pad[half:])
    zeros_blk = jnp.zeros((np_ // 16, D), F32)

    def unpack(p, split_first=True):
        lin = p["lin"]
        out = []
        for i, l in enumerate(lin):
            out.append(_split_w(l["w"]) if (split_first or i > 0) else l["w"])
            out.append(l["b"].reshape(1, -1))
        if "ln" in p:
            out.append(p["ln"]["g"].reshape(1, -1))
            out.append(p["ln"]["b"].reshape(1, -1))
        return out

    h = _node_enc(x_pad, *unpack(params["node_enc"]))
    enc_w = unpack(params["edge_enc"], split_first=False)
    e = [_edge_enc(ea_c, *enc_w) for ea_c in ea_chunks]

    for blk in params["blocks"]:
        w1 = blk["edge_mlp"]["lin"][0]["w"]
        ew = unpack(blk["edge_mlp"])[1:]
        w1cp = _split_w(w1[2 * D :])
        u, v = _uv(h, _split_w(w1[:D]), _split_w(w1[D : 2 * D]))
        gath = [_sc_gather(u, v, schunks[c], dchunks[c]) for c in range(2)]
        e = [
            _edge_mlp(gath[c][0], gath[c][1], e[c], ew[0], w1cp, *ew[1:])
            for c in range(2)
        ]
        parts = _sc_scatter2(e[0], e[1], dchunks[0], dchunks[1], zeros_blk)
        w1n = blk["node_mlp"]["lin"][0]["w"]
        nw = unpack(blk["node_mlp"])[1:]
        h = _node_mlp(h, parts, _split_w(w1n[:D]), _split_w(w1n[D:]), *nw)

    dw = unpack(params["node_dec"])
    w3 = params["node_dec"]["lin"][2]["w"]
    out_dim = w3.shape[1]
    w3p = _split_w(jnp.pad(w3, ((0, 0), (0, D - out_dim))))
    b3p = jnp.pad(dw[5], ((0, 0), (0, D - out_dim)))
    out = _decoder(h, dw[0], dw[1], dw[2], dw[3], w3p, b3p)
    return out[:n, :out_dim]


# speedup vs baseline: 1.3840x; 1.1872x over previous
import functools

import jax
import jax.numpy as jnp
from jax import lax
from jax.experimental import pallas as pl
from jax.experimental.pallas import tpu as pltpu
from jax.experimental.pallas import tpu_sc as plsc

F32 = jnp.float32
BF16 = jnp.bfloat16
D = 128
BLK = 2048


def _round_up(v, m):
    return (v + m - 1) // m * m


def _row(i):
    return (i, 0)


def _cst(i):
    return (0, 0)


def _cst3(i):
    return (0, 0, 0)


def _full_specs(*arrs):
    return [pl.BlockSpec(a.shape, _cst3 if a.ndim == 3 else _cst) for a in arrs]


def _ln(xv, g, b):
    mu = jnp.mean(xv, axis=-1, keepdims=True)
    xc = xv - mu
    var = jnp.mean(xc * xc, axis=-1, keepdims=True)
    return xc * lax.rsqrt(var + 1e-5) * g + b


def _dot3(a, wp):
    ah = a.astype(BF16)
    al = (a - ah.astype(F32)).astype(BF16)
    a3 = jnp.concatenate([ah, al, ah], axis=1)
    return jnp.dot(a3, wp[...], preferred_element_type=F32)


def _split_w(w):
    hi = w.astype(BF16)
    lo = (w - hi.astype(F32)).astype(BF16)
    return jnp.concatenate([hi, hi, lo], axis=0)


def _node_enc(x, w1, b1, w2, b2, w3, b3, g, b):
    n = x.shape[0]
    blk = min(BLK, n)

    def body(x_ref, w1r, b1r, w2r, b2r, w3r, b3r, gr, br, o_ref):
        v = jnp.maximum(_dot3(x_ref[...], w1r) + b1r[...], 0.0)
        v = jnp.maximum(_dot3(v, w2r) + b2r[...], 0.0)
        v = _dot3(v, w3r) + b3r[...]
        o_ref[...] = _ln(v, gr[...], br[...])

    return pl.pallas_call(
        body,
        grid=(n // blk,),
        in_specs=[pl.BlockSpec((blk, x.shape[1]), _row)] + _full_specs(w1, b1, w2, b2, w3, b3, g, b),
        out_specs=pl.BlockSpec((blk, D), _row),
        out_shape=jax.ShapeDtypeStruct((n, D), F32),
    )(x, w1, b1, w2, b2, w3, b3, g, b)


def _edge_enc(ea, w1, b1, w2, b2, w3, b3, g, b):
    n, din = ea.shape

    def body(ea_ref, w1r, b1r, w2r, b2r, w3r, b3r, gr, br, o_ref):
        acc = jnp.broadcast_to(b1r[...], (BLK, D))
        for k in range(din):
            acc = acc + ea_ref[:, k : k + 1] * w1r[k : k + 1, :]
        v = jnp.maximum(acc, 0.0)
        v = jnp.maximum(_dot3(v, w2r) + b2r[...], 0.0)
        v = _dot3(v, w3r) + b3r[...]
        o_ref[...] = _ln(v, gr[...], br[...])

    return pl.pallas_call(
        body,
        grid=(n // BLK,),
        in_specs=[pl.BlockSpec((BLK, din), _row)] + _full_specs(w1, b1, w2, b2, w3, b3, g, b),
        out_specs=pl.BlockSpec((BLK, D), _row),
        out_shape=jax.ShapeDtypeStruct((n, D), F32),
    )(ea, w1, b1, w2, b2, w3, b3, g, b)


def _uv(h, w1a, w1b):
    n = h.shape[0]
    blk = min(BLK, n)

    def body(h_ref, war, wbr, u_ref, v_ref):
        u_ref[...] = _dot3(h_ref[...], war)
        v_ref[...] = _dot3(h_ref[...], wbr)

    return pl.pallas_call(
        body,
        grid=(n // blk,),
        in_specs=[pl.BlockSpec((blk, D), _row)] + _full_specs(w1a, w1b),
        out_specs=[pl.BlockSpec((blk, D), _row)] * 2,
        out_shape=[jax.ShapeDtypeStruct((n, D), F32)] * 2,
    )(h, w1a, w1b)


def _edge_mlp(us, vd, e, b1, w1c, w2, b2, w3, b3, g, b):
    n = e.shape[0]

    def body(us_ref, vd_ref, e_ref, b1r, w1cr, w2r, b2r, w3r, b3r, gr, br, o_ref):
        v = us_ref[...] + vd_ref[...] + _dot3(e_ref[...], w1cr) + b1r[...]
        v = jnp.maximum(v, 0.0)
        v = jnp.maximum(_dot3(v, w2r) + b2r[...], 0.0)
        v = _dot3(v, w3r) + b3r[...]
        o_ref[...] = e_ref[...] + _ln(v, gr[...], br[...])

    return pl.pallas_call(
        body,
        grid=(n // BLK,),
        in_specs=[pl.BlockSpec((BLK, D), _row)] * 3 + _full_specs(b1, w1c, w2, b2, w3, b3, g, b),
        out_specs=pl.BlockSpec((BLK, D), _row),
        out_shape=jax.ShapeDtypeStruct((n, D), F32),
    )(us, vd, e, b1, w1c, w2, b2, w3, b3, g, b)


def _node_mlp(h, parts, w1h, w1a, b1, w2, b2, w3, b3, g, b):
    n = h.shape[0]
    blk = min(BLK, n)

    def body(h_ref, p_ref, w1hr, w1ar, b1r, w2r, b2r, w3r, b3r, gr, br, o_ref):
        agg = p_ref[0] + p_ref[1]
        v = _dot3(h_ref[...], w1hr) + _dot3(agg, w1ar) + b1r[...]
        v = jnp.maximum(v, 0.0)
        v = jnp.maximum(_dot3(v, w2r) + b2r[...], 0.0)
        v = _dot3(v, w3r) + b3r[...]
        o_ref[...] = h_ref[...] + _ln(v, gr[...], br[...])

    return pl.pallas_call(
        body,
        grid=(n // blk,),
        in_specs=[pl.BlockSpec((blk, D), _row), pl.BlockSpec((2, blk, D), lambda i: (0, i, 0))]
        + _full_specs(w1h, w1a, b1, w2, b2, w3, b3, g, b),
        out_specs=pl.BlockSpec((blk, D), _row),
        out_shape=jax.ShapeDtypeStruct((n, D), F32),
    )(h, parts, w1h, w1a, b1, w2, b2, w3, b3, g, b)


def _decoder(h, w1, b1, w2, b2, w3, b3):
    n = h.shape[0]
    blk = min(BLK, n)

    def body(h_ref, w1r, b1r, w2r, b2r, w3r, b3r, o_ref):
        v = jnp.maximum(_dot3(h_ref[...], w1r) + b1r[...], 0.0)
        v = jnp.maximum(_dot3(v, w2r) + b2r[...], 0.0)
        o_ref[...] = _dot3(v, w3r) + b3r[...]

    return pl.pallas_call(
        body,
        grid=(n // blk,),
        in_specs=[pl.BlockSpec((blk, D), _row)] + _full_specs(w1, b1, w2, b2, w3, b3),
        out_specs=pl.BlockSpec((blk, D), _row),
        out_shape=jax.ShapeDtypeStruct((n, D), F32),
    )(h, w1, b1, w2, b2, w3, b3)


def _sc_gather(u, v, src2d, dst2d):
    nb = src2d.shape[0]
    ep = nb * 128
    np_ = u.shape[0]
    rows = np_ // 16
    mesh = plsc.VectorSubcoreMesh(core_axis_name="c", subcore_axis_name="s")

    @functools.partial(
        pl.kernel,
        mesh=mesh,
        out_type=[
            jax.ShapeDtypeStruct((ep, D), F32),
            jax.ShapeDtypeStruct((ep, D), F32),
        ],
        scratch_types=[pltpu.VMEM_SHARED((np_, D), F32)],
    )
    def gk(u_hbm, v_hbm, s_hbm, d_hbm, us_hbm, vd_hbm, table_sh):
        cid = lax.axis_index("c")
        sid = lax.axis_index("s")
        sl = pl.ds(sid * rows, rows)

        @pl.when(cid == 0)
        def _():
            pltpu.sync_copy(u_hbm.at[sl], table_sh.at[sl])

        @pl.when(cid == 1)
        def _():
            pltpu.sync_copy(v_hbm.at[sl], table_sh.at[sl])

        plsc.subcore_barrier()

        def body(i_vmem, o_vmem):
            pltpu.sync_copy(table_sh.at[i_vmem.at[0]], o_vmem)

        pipe = functools.partial(
            pltpu.emit_pipeline,
            body,
            grid=(nb,),
            in_specs=[pl.BlockSpec((1, 128), _row)],
            out_specs=[pl.BlockSpec((128, D), _row)],
            core_axis_name="s",
            dimension_semantics=(pltpu.PARALLEL,),
        )

        @pl.when(cid == 0)
        def _():
            pipe()(s_hbm, us_hbm)

        @pl.when(cid == 1)
        def _():
            pipe()(d_hbm, vd_hbm)

    return gk(u, v, src2d, dst2d)


def _sc_gather_hbm(u, v, src2d, dst2d):
    nb = src2d.shape[0]
    ep = nb * 128
    mesh = plsc.VectorSubcoreMesh(core_axis_name="c", subcore_axis_name="s")

    @functools.partial(
        pl.kernel,
        mesh=mesh,
        out_type=[
            jax.ShapeDtypeStruct((ep, D), F32),
            jax.ShapeDtypeStruct((ep, D), F32),
        ],
    )
    def gk(u_hbm, v_hbm, s_hbm, d_hbm, us_hbm, vd_hbm):
        def body(s_vmem, d_vmem, us_vmem, vd_vmem):
            pltpu.sync_copy(u_hbm.at[s_vmem.at[0]], us_vmem)
            pltpu.sync_copy(v_hbm.at[d_vmem.at[0]], vd_vmem)

        pltpu.emit_pipeline(
            body,
            grid=(nb,),
            in_specs=[
                pl.BlockSpec((1, 128), _row),
                pl.BlockSpec((1, 128), _row),
            ],
            out_specs=[
                pl.BlockSpec((128, D), _row),
                pl.BlockSpec((128, D), _row),
            ],
            core_axis_name=("c", "s"),
            dimension_semantics=(pltpu.PARALLEL,),
        )(s_hbm, d_hbm, us_hbm, vd_hbm)

    return gk(u, v, src2d, dst2d)


def _sc_scatter(e_new, dst2d, zeros_blk):
    nb = dst2d.shape[0]
    rows = zeros_blk.shape[0]
    np_ = rows * 16
    mesh = plsc.VectorSubcoreMesh(core_axis_name="c", subcore_axis_name="s")

    @functools.partial(
        pl.kernel,
        mesh=mesh,
        out_type=jax.ShapeDtypeStruct((2, np_, D), F32),
        scratch_types=[pltpu.VMEM_SHARED((np_, D), F32)],
    )
    def sk(e_hbm, d_hbm, z_hbm, out_hbm, acc_shared):
        cid = lax.axis_index("c")
        sid = lax.axis_index("s")
        pltpu.sync_copy(z_hbm, acc_shared.at[pl.ds(sid * rows, rows)])
        plsc.subcore_barrier()

        def body(e_vmem, d_vmem):
            pltpu.sync_copy(e_vmem, acc_shared.at[d_vmem.at[0]], add=True)

        pltpu.emit_pipeline(
            body,
            grid=(nb,),
            in_specs=[
                pl.BlockSpec((128, D), _row),
                pl.BlockSpec((1, 128), _row),
            ],
            out_specs=[],
            core_axis_name=("c", "s"),
            dimension_semantics=(pltpu.PARALLEL,),
        )(e_hbm, d_hbm)

        plsc.subcore_barrier()
        pltpu.sync_copy(
            acc_shared.at[pl.ds(sid * rows, rows)],
            out_hbm.at[cid].at[pl.ds(sid * rows, rows)],
        )

    return sk(e_new, dst2d, zeros_blk)


def _sc_scatter2(e0, e1, d0, d1, zeros_blk):
    nb = d0.shape[0]
    rows = zeros_blk.shape[0]
    np_ = rows * 16
    mesh = plsc.VectorSubcoreMesh(core_axis_name="c", subcore_axis_name="s")

    @functools.partial(
        pl.kernel,
        mesh=mesh,
        out_type=jax.ShapeDtypeStruct((2, np_, D), F32),
        scratch_types=[pltpu.VMEM_SHARED((np_, D), F32)],
    )
    def sk(e0_hbm, e1_hbm, d0_hbm, d1_hbm, z_hbm, out_hbm, acc_shared):
        cid = lax.axis_index("c")
        sid = lax.axis_index("s")
        pltpu.sync_copy(z_hbm, acc_shared.at[pl.ds(sid * rows, rows)])
        plsc.subcore_barrier()

        def body(e_vmem, d_vmem):
            pltpu.sync_copy(e_vmem, acc_shared.at[d_vmem.at[0]], add=True)

        for e_hbm, d_hbm in ((e0_hbm, d0_hbm), (e1_hbm, d1_hbm)):
            pltpu.emit_pipeline(
                body,
                grid=(nb,),
                in_specs=[
                    pl.BlockSpec((128, D), _row),
                    pl.BlockSpec((1, 128), _row),
                ],
                out_specs=[],
                core_axis_name=("c", "s"),
                dimension_semantics=(pltpu.PARALLEL,),
            )(e_hbm, d_hbm)

        plsc.subcore_barrier()
        pltpu.sync_copy(
            acc_shared.at[pl.ds(sid * rows, rows)],
            out_hbm.at[cid].at[pl.ds(sid * rows, rows)],
        )

    return sk(e0, e1, d0, d1, zeros_blk)


def kernel(x, edge_index, edge_attr, params):
    n = x.shape[0]
    ne = edge_attr.shape[0]
    np_ = _round_up(n, 2048)
    ep = _round_up(ne, 8192)
    half = ep // 2
    hb = half // 128

    src = edge_index[0].astype(jnp.int32)
    dst = edge_index[1].astype(jnp.int32)
    pad_ids = (jnp.arange(ep - ne, dtype=jnp.int32) % (np_ - n)) + n
    src2d = jnp.concatenate([src, pad_ids]).reshape(ep // 128, 128)
    dst2d = jnp.concatenate([dst, pad_ids]).reshape(ep // 128, 128)
    schunks = (src2d[:hb], src2d[hb:])
    dchunks = (dst2d[:hb], dst2d[hb:])

    x_pad = jnp.pad(x, ((0, np_ - n), (0, 0)))
    ea_pad = jnp.pad(edge_attr, ((0, ep - ne), (0, 0)))
    ea_chunks = (ea_pad[:half], ea_pad[half:])
    zeros_blk = jnp.zeros((np_ // 16, D), F32)

    def unpack(p, split_first=True):
        lin = p["lin"]
        out = []
        for i, l in enumerate(lin):
            out.append(_split_w(l["w"]) if (split_first or i > 0) else l["w"])
            out.append(l["b"].reshape(1, -1))
        if "ln" in p:
            out.append(p["ln"]["g"].reshape(1, -1))
            out.append(p["ln"]["b"].reshape(1, -1))
        return out

    h = _node_enc(x_pad, *unpack(params["node_enc"]))
    enc_w = unpack(params["edge_enc"], split_first=False)
    e = [_edge_enc(ea_c, *enc_w) for ea_c in ea_chunks]

    for blk in params["blocks"]:
        w1 = blk["edge_mlp"]["lin"][0]["w"]
        ew = unpack(blk["edge_mlp"])[1:]
        w1cp = _split_w(w1[2 * D :])
        u, v = _uv(h, _split_w(w1[:D]), _split_w(w1[D : 2 * D]))
        gath = [_sc_gather(u, v, schunks[c], dchunks[c]) for c in range(2)]
        e = [
            _edge_mlp(gath[c][0], gath[c][1], e[c], ew[0], w1cp, *ew[1:])
            for c in range(2)
        ]
        parts = _sc_scatter2(e[0], e[1], dchunks[0], dchunks[1], zeros_blk)
        w1n = blk["node_mlp"]["lin"][0]["w"]
        nw = unpack(blk["node_mlp"])[1:]
        h = _node_mlp(h, parts, _split_w(w1n[:D]), _split_w(w1n[D:]), *nw)

    dw = unpack(params["node_dec"])
    w3 = params["node_dec"]["lin"][2]["w"]
    out_dim = w3.shape[1]
    w3p = _split_w(jnp.pad(w3, ((0, 0), (0, D - out_dim))))
    b3p = jnp.pad(dw[5], ((0, 0), (0, D - out_dim)))
    out = _decoder(h, dw[0], dw[1], dw[2], dw[3], w3p, b3p)
    return out[:n, :out_dim]


# speedup vs baseline: 1.5022x; 1.0854x over previous
import functools

import jax
import jax.numpy as jnp
from jax import lax
from jax.experimental import pallas as pl
from jax.experimental.pallas import tpu as pltpu
from jax.experimental.pallas import tpu_sc as plsc

F32 = jnp.float32
BF16 = jnp.bfloat16
D = 128
BLK = 2048
EBLK = 4096


def _round_up(v, m):
    return (v + m - 1) // m * m


def _row(i):
    return (i, 0)


def _cst(i):
    return (0, 0)


def _cst3(i):
    return (0, 0, 0)


def _full_specs(*arrs):
    return [pl.BlockSpec(a.shape, _cst3 if a.ndim == 3 else _cst) for a in arrs]


def _ln(xv, g, b):
    mu = jnp.mean(xv, axis=-1, keepdims=True)
    xc = xv - mu
    var = jnp.mean(xc * xc, axis=-1, keepdims=True)
    return xc * lax.rsqrt(var + 1e-5) * g + b


def _dot3(a, wp):
    ah = a.astype(BF16)
    al = (a - ah.astype(F32)).astype(BF16)
    a3 = jnp.concatenate([ah, al, ah], axis=1)
    return jnp.dot(a3, wp[...], preferred_element_type=F32)


def _split_w(w):
    hi = w.astype(BF16)
    lo = (w - hi.astype(F32)).astype(BF16)
    return jnp.concatenate([hi, hi, lo], axis=0)


def _node_enc(x, w1, b1, w2, b2, w3, b3, g, b):
    n = x.shape[0]
    blk = min(BLK, n)

    def body(x_ref, w1r, b1r, w2r, b2r, w3r, b3r, gr, br, o_ref):
        v = jnp.maximum(_dot3(x_ref[...], w1r) + b1r[...], 0.0)
        v = jnp.maximum(_dot3(v, w2r) + b2r[...], 0.0)
        v = _dot3(v, w3r) + b3r[...]
        o_ref[...] = _ln(v, gr[...], br[...])

    return pl.pallas_call(
        body,
        grid=(n // blk,),
        in_specs=[pl.BlockSpec((blk, x.shape[1]), _row)] + _full_specs(w1, b1, w2, b2, w3, b3, g, b),
        out_specs=pl.BlockSpec((blk, D), _row),
        out_shape=jax.ShapeDtypeStruct((n, D), F32),
    )(x, w1, b1, w2, b2, w3, b3, g, b)


def _edge_enc(ea, w1, b1, w2, b2, w3, b3, g, b):
    n, din = ea.shape

    def body(ea_ref, w1r, b1r, w2r, b2r, w3r, b3r, gr, br, o_ref):
        acc = jnp.broadcast_to(b1r[...], (EBLK, D))
        for k in range(din):
            acc = acc + ea_ref[:, k : k + 1] * w1r[k : k + 1, :]
        v = jnp.maximum(acc, 0.0)
        v = jnp.maximum(_dot3(v, w2r) + b2r[...], 0.0)
        v = _dot3(v, w3r) + b3r[...]
        o_ref[...] = _ln(v, gr[...], br[...])

    return pl.pallas_call(
        body,
        grid=(n // EBLK,),
        in_specs=[pl.BlockSpec((EBLK, din), _row)] + _full_specs(w1, b1, w2, b2, w3, b3, g, b),
        out_specs=pl.BlockSpec((EBLK, D), _row),
        out_shape=jax.ShapeDtypeStruct((n, D), F32),
    )(ea, w1, b1, w2, b2, w3, b3, g, b)


def _uv(h, w1a, w1b):
    n = h.shape[0]
    blk = min(BLK, n)

    def body(h_ref, war, wbr, u_ref, v_ref):
        u_ref[...] = _dot3(h_ref[...], war)
        v_ref[...] = _dot3(h_ref[...], wbr)

    return pl.pallas_call(
        body,
        grid=(n // blk,),
        in_specs=[pl.BlockSpec((blk, D), _row)] + _full_specs(w1a, w1b),
        out_specs=[pl.BlockSpec((blk, D), _row)] * 2,
        out_shape=[jax.ShapeDtypeStruct((n, D), F32)] * 2,
    )(h, w1a, w1b)


def _edge_mlp(us, vd, e, b1, w1c, w2, b2, w3, b3, g, b):
    n = e.shape[0]

    def body(us_ref, vd_ref, e_ref, b1r, w1cr, w2r, b2r, w3r, b3r, gr, br, o_ref):
        v = us_ref[...] + vd_ref[...] + _dot3(e_ref[...], w1cr) + b1r[...]
        v = jnp.maximum(v, 0.0)
        v = jnp.maximum(_dot3(v, w2r) + b2r[...], 0.0)
        v = _dot3(v, w3r) + b3r[...]
        o_ref[...] = e_ref[...] + _ln(v, gr[...], br[...])

    return pl.pallas_call(
        body,
        grid=(n // EBLK,),
        in_specs=[pl.BlockSpec((EBLK, D), _row)] * 3 + _full_specs(b1, w1c, w2, b2, w3, b3, g, b),
        out_specs=pl.BlockSpec((EBLK, D), _row),
        out_shape=jax.ShapeDtypeStruct((n, D), F32),
    )(us, vd, e, b1, w1c, w2, b2, w3, b3, g, b)


def _node_mlp(h, parts, w1h, w1a, b1, w2, b2, w3, b3, g, b):
    n = h.shape[0]
    blk = min(BLK, n)

    def body(h_ref, p_ref, w1hr, w1ar, b1r, w2r, b2r, w3r, b3r, gr, br, o_ref):
        agg = p_ref[0] + p_ref[1]
        v = _dot3(h_ref[...], w1hr) + _dot3(agg, w1ar) + b1r[...]
        v = jnp.maximum(v, 0.0)
        v = jnp.maximum(_dot3(v, w2r) + b2r[...], 0.0)
        v = _dot3(v, w3r) + b3r[...]
        o_ref[...] = h_ref[...] + _ln(v, gr[...], br[...])

    return pl.pallas_call(
        body,
        grid=(n // blk,),
        in_specs=[pl.BlockSpec((blk, D), _row), pl.BlockSpec((2, blk, D), lambda i: (0, i, 0))]
        + _full_specs(w1h, w1a, b1, w2, b2, w3, b3, g, b),
        out_specs=pl.BlockSpec((blk, D), _row),
        out_shape=jax.ShapeDtypeStruct((n, D), F32),
    )(h, parts, w1h, w1a, b1, w2, b2, w3, b3, g, b)


def _decoder(h, w1, b1, w2, b2, w3, b3):
    n = h.shape[0]
    blk = min(BLK, n)

    def body(h_ref, w1r, b1r, w2r, b2r, w3r, b3r, o_ref):
        v = jnp.maximum(_dot3(h_ref[...], w1r) + b1r[...], 0.0)
        v = jnp.maximum(_dot3(v, w2r) + b2r[...], 0.0)
        o_ref[...] = _dot3(v, w3r) + b3r[...]

    return pl.pallas_call(
        body,
        grid=(n // blk,),
        in_specs=[pl.BlockSpec((blk, D), _row)] + _full_specs(w1, b1, w2, b2, w3, b3),
        out_specs=pl.BlockSpec((blk, D), _row),
        out_shape=jax.ShapeDtypeStruct((n, D), F32),
    )(h, w1, b1, w2, b2, w3, b3)


def _sc_gather(u, v, src2d, dst2d):
    nb = src2d.shape[0]
    ep = nb * 128
    np_ = u.shape[0]
    rows = np_ // 16
    mesh = plsc.VectorSubcoreMesh(core_axis_name="c", subcore_axis_name="s")

    @functools.partial(
        pl.kernel,
        mesh=mesh,
        out_type=[
            jax.ShapeDtypeStruct((ep, D), F32),
            jax.ShapeDtypeStruct((ep, D), F32),
        ],
        scratch_types=[pltpu.VMEM_SHARED((np_, D), F32)],
    )
    def gk(u_hbm, v_hbm, s_hbm, d_hbm, us_hbm, vd_hbm, table_sh):
        cid = lax.axis_index("c")
        sid = lax.axis_index("s")
        sl = pl.ds(sid * rows, rows)

        @pl.when(cid == 0)
        def _():
            pltpu.sync_copy(u_hbm.at[sl], table_sh.at[sl])

        @pl.when(cid == 1)
        def _():
            pltpu.sync_copy(v_hbm.at[sl], table_sh.at[sl])

        plsc.subcore_barrier()

        def body(i_vmem, o_vmem):
            pltpu.sync_copy(table_sh.at[i_vmem.at[0]], o_vmem)

        pipe = functools.partial(
            pltpu.emit_pipeline,
            body,
            grid=(nb,),
            in_specs=[pl.BlockSpec((1, 128), _row)],
            out_specs=[pl.BlockSpec((128, D), _row)],
            core_axis_name="s",
            dimension_semantics=(pltpu.PARALLEL,),
        )

        @pl.when(cid == 0)
        def _():
            pipe()(s_hbm, us_hbm)

        @pl.when(cid == 1)
        def _():
            pipe()(d_hbm, vd_hbm)

    return gk(u, v, src2d, dst2d)


def _sc_gather_hbm(u, v, src2d, dst2d):
    nb = src2d.shape[0]
    ep = nb * 128
    mesh = plsc.VectorSubcoreMesh(core_axis_name="c", subcore_axis_name="s")

    @functools.partial(
        pl.kernel,
        mesh=mesh,
        out_type=[
            jax.ShapeDtypeStruct((ep, D), F32),
            jax.ShapeDtypeStruct((ep, D), F32),
        ],
    )
    def gk(u_hbm, v_hbm, s_hbm, d_hbm, us_hbm, vd_hbm):
        def body(s_vmem, d_vmem, us_vmem, vd_vmem):
            pltpu.sync_copy(u_hbm.at[s_vmem.at[0]], us_vmem)
            pltpu.sync_copy(v_hbm.at[d_vmem.at[0]], vd_vmem)

        pltpu.emit_pipeline(
            body,
            grid=(nb,),
            in_specs=[
                pl.BlockSpec((1, 128), _row),
                pl.BlockSpec((1, 128), _row),
            ],
            out_specs=[
                pl.BlockSpec((128, D), _row),
                pl.BlockSpec((128, D), _row),
            ],
            core_axis_name=("c", "s"),
            dimension_semantics=(pltpu.PARALLEL,),
        )(s_hbm, d_hbm, us_hbm, vd_hbm)

    return gk(u, v, src2d, dst2d)


def _sc_scatter(e_new, dst2d, zeros_blk):
    nb = dst2d.shape[0]
    rows = zeros_blk.shape[0]
    np_ = rows * 16
    mesh = plsc.VectorSubcoreMesh(core_axis_name="c", subcore_axis_name="s")

    @functools.partial(
        pl.kernel,
        mesh=mesh,
        out_type=jax.ShapeDtypeStruct((2, np_, D), F32),
        scratch_types=[pltpu.VMEM_SHARED((np_, D), F32)],
    )
    def sk(e_hbm, d_hbm, z_hbm, out_hbm, acc_shared):
        cid = lax.axis_index("c")
        sid = lax.axis_index("s")
        pltpu.sync_copy(z_hbm, acc_shared.at[pl.ds(sid * rows, rows)])
        plsc.subcore_barrier()

        def body(e_vmem, d_vmem):
            pltpu.sync_copy(e_vmem, acc_shared.at[d_vmem.at[0]], add=True)

        pltpu.emit_pipeline(
            body,
            grid=(nb,),
            in_specs=[
                pl.BlockSpec((128, D), _row),
                pl.BlockSpec((1, 128), _row),
            ],
            out_specs=[],
            core_axis_name=("c", "s"),
            dimension_semantics=(pltpu.PARALLEL,),
        )(e_hbm, d_hbm)

        plsc.subcore_barrier()
        pltpu.sync_copy(
            acc_shared.at[pl.ds(sid * rows, rows)],
            out_hbm.at[cid].at[pl.ds(sid * rows, rows)],
        )

    return sk(e_new, dst2d, zeros_blk)


def _sc_scatter2(e0, e1, d0, d1, zeros_blk):
    nb = d0.shape[0]
    rows = zeros_blk.shape[0]
    np_ = rows * 16
    mesh = plsc.VectorSubcoreMesh(core_axis_name="c", subcore_axis_name="s")

    @functools.partial(
        pl.kernel,
        mesh=mesh,
        out_type=jax.ShapeDtypeStruct((2, np_, D), F32),
        scratch_types=[pltpu.VMEM_SHARED((np_, D), F32)],
    )
    def sk(e0_hbm, e1_hbm, d0_hbm, d1_hbm, z_hbm, out_hbm, acc_shared):
        cid = lax.axis_index("c")
        sid = lax.axis_index("s")
        pltpu.sync_copy(z_hbm, acc_shared.at[pl.ds(sid * rows, rows)])
        plsc.subcore_barrier()

        def body(e_vmem, d_vmem):
            pltpu.sync_copy(e_vmem, acc_shared.at[d_vmem.at[0]], add=True)

        for e_hbm, d_hbm in ((e0_hbm, d0_hbm), (e1_hbm, d1_hbm)):
            pltpu.emit_pipeline(
                body,
                grid=(nb,),
                in_specs=[
                    pl.BlockSpec((128, D), _row),
                    pl.BlockSpec((1, 128), _row),
                ],
                out_specs=[],
                core_axis_name=("c", "s"),
                dimension_semantics=(pltpu.PARALLEL,),
            )(e_hbm, d_hbm)

        plsc.subcore_barrier()
        pltpu.sync_copy(
            acc_shared.at[pl.ds(sid * rows, rows)],
            out_hbm.at[cid].at[pl.ds(sid * rows, rows)],
        )

    return sk(e0, e1, d0, d1, zeros_blk)


def kernel(x, edge_index, edge_attr, params):
    n = x.shape[0]
    ne = edge_attr.shape[0]
    np_ = _round_up(n, 2048)
    ep = _round_up(ne, 8192)
    half = ep // 2
    hb = half // 128

    src = edge_index[0].astype(jnp.int32)
    dst = edge_index[1].astype(jnp.int32)
    pad_ids = (jnp.arange(ep - ne, dtype=jnp.int32) % (np_ - n)) + n
    src2d = jnp.concatenate([src, pad_ids]).reshape(ep // 128, 128)
    dst2d = jnp.concatenate([dst, pad_ids]).reshape(ep // 128, 128)
    schunks = (src2d[:hb], src2d[hb:])
    dchunks = (dst2d[:hb], dst2d[hb:])

    x_pad = jnp.pad(x, ((0, np_ - n), (0, 0)))
    ea_pad = jnp.pad(edge_attr, ((0, ep - ne), (0, 0)))
    ea_chunks = (ea_pad[:half], ea_pad[half:])
    zeros_blk = jnp.zeros((np_ // 16, D), F32)

    def unpack(p, split_first=True):
        lin = p["lin"]
        out = []
        for i, l in enumerate(lin):
            out.append(_split_w(l["w"]) if (split_first or i > 0) else l["w"])
            out.append(l["b"].reshape(1, -1))
        if "ln" in p:
            out.append(p["ln"]["g"].reshape(1, -1))
            out.append(p["ln"]["b"].reshape(1, -1))
        return out

    h = _node_enc(x_pad, *unpack(params["node_enc"]))
    enc_w = unpack(params["edge_enc"], split_first=False)
    e = [_edge_enc(ea_c, *enc_w) for ea_c in ea_chunks]

    for blk in params["blocks"]:
        w1 = blk["edge_mlp"]["lin"][0]["w"]
        ew = unpack(blk["edge_mlp"])[1:]
        w1cp = _split_w(w1[2 * D :])
        u, v = _uv(h, _split_w(w1[:D]), _split_w(w1[D : 2 * D]))
        gath = [_sc_gather(u, v, schunks[c], dchunks[c]) for c in range(2)]
        e = [
            _edge_mlp(gath[c][0], gath[c][1], e[c], ew[0], w1cp, *ew[1:])
            for c in range(2)
        ]
        parts = _sc_scatter2(e[0], e[1], dchunks[0], dchunks[1], zeros_blk)
        w1n = blk["node_mlp"]["lin"][0]["w"]
        nw = unpack(blk["node_mlp"])[1:]
        h = _node_mlp(h, parts, _split_w(w1n[:D]), _split_w(w1n[D:]), *nw)

    dw = unpack(params["node_dec"])
    w3 = params["node_dec"]["lin"][2]["w"]
    out_dim = w3.shape[1]
    w3p = _split_w(jnp.pad(w3, ((0, 0), (0, D - out_dim))))
    b3p = jnp.pad(dw[5], ((0, 0), (0, D - out_dim)))
    out = _decoder(h, dw[0], dw[1], dw[2], dw[3], w3p, b3p)
    return out[:n, :out_dim]


# speedup vs baseline: 1.5490x; 1.0311x over previous
import functools

import jax
import jax.numpy as jnp
from jax import lax
from jax.experimental import pallas as pl
from jax.experimental.pallas import tpu as pltpu
from jax.experimental.pallas import tpu_sc as plsc

F32 = jnp.float32
BF16 = jnp.bfloat16
D = 128
BLK = 2048
EBLK = 8192


def _round_up(v, m):
    return (v + m - 1) // m * m


def _row(i):
    return (i, 0)


def _cst(i):
    return (0, 0)


def _cst3(i):
    return (0, 0, 0)


def _full_specs(*arrs):
    return [pl.BlockSpec(a.shape, _cst3 if a.ndim == 3 else _cst) for a in arrs]


def _ln(xv, g, b):
    mu = jnp.mean(xv, axis=-1, keepdims=True)
    xc = xv - mu
    var = jnp.mean(xc * xc, axis=-1, keepdims=True)
    return xc * lax.rsqrt(var + 1e-5) * g + b


def _dot3(a, wp):
    ah = a.astype(BF16)
    al = (a - ah.astype(F32)).astype(BF16)
    a3 = jnp.concatenate([ah, al, ah], axis=1)
    return jnp.dot(a3, wp[...], preferred_element_type=F32)


def _split_w(w):
    hi = w.astype(BF16)
    lo = (w - hi.astype(F32)).astype(BF16)
    return jnp.concatenate([hi, hi, lo], axis=0)


def _node_enc(x, w1, b1, w2, b2, w3, b3, g, b):
    n = x.shape[0]
    blk = min(BLK, n)

    def body(x_ref, w1r, b1r, w2r, b2r, w3r, b3r, gr, br, o_ref):
        v = jnp.maximum(_dot3(x_ref[...], w1r) + b1r[...], 0.0)
        v = jnp.maximum(_dot3(v, w2r) + b2r[...], 0.0)
        v = _dot3(v, w3r) + b3r[...]
        o_ref[...] = _ln(v, gr[...], br[...])

    return pl.pallas_call(
        body,
        grid=(n // blk,),
        in_specs=[pl.BlockSpec((blk, x.shape[1]), _row)] + _full_specs(w1, b1, w2, b2, w3, b3, g, b),
        out_specs=pl.BlockSpec((blk, D), _row),
        out_shape=jax.ShapeDtypeStruct((n, D), F32),
    )(x, w1, b1, w2, b2, w3, b3, g, b)


def _edge_enc(ea, w1, b1, w2, b2, w3, b3, g, b):
    n, din = ea.shape

    def body(ea_ref, w1r, b1r, w2r, b2r, w3r, b3r, gr, br, o_ref):
        acc = jnp.broadcast_to(b1r[...], (EBLK, D))
        for k in range(din):
            acc = acc + ea_ref[:, k : k + 1] * w1r[k : k + 1, :]
        v = jnp.maximum(acc, 0.0)
        v = jnp.maximum(_dot3(v, w2r) + b2r[...], 0.0)
        v = _dot3(v, w3r) + b3r[...]
        o_ref[...] = _ln(v, gr[...], br[...])

    return pl.pallas_call(
        body,
        grid=(n // EBLK,),
        in_specs=[pl.BlockSpec((EBLK, din), _row)] + _full_specs(w1, b1, w2, b2, w3, b3, g, b),
        out_specs=pl.BlockSpec((EBLK, D), _row),
        out_shape=jax.ShapeDtypeStruct((n, D), F32),
    )(ea, w1, b1, w2, b2, w3, b3, g, b)


def _uv(h, w1a, w1b):
    n = h.shape[0]
    blk = min(BLK, n)

    def body(h_ref, war, wbr, u_ref, v_ref):
        u_ref[...] = _dot3(h_ref[...], war)
        v_ref[...] = _dot3(h_ref[...], wbr)

    return pl.pallas_call(
        body,
        grid=(n // blk,),
        in_specs=[pl.BlockSpec((blk, D), _row)] + _full_specs(w1a, w1b),
        out_specs=[pl.BlockSpec((blk, D), _row)] * 2,
        out_shape=[jax.ShapeDtypeStruct((n, D), F32)] * 2,
    )(h, w1a, w1b)


def _edge_mlp(us, vd, e, b1, w1c, w2, b2, w3, b3, g, b):
    n = e.shape[0]

    def body(us_ref, vd_ref, e_ref, b1r, w1cr, w2r, b2r, w3r, b3r, gr, br, o_ref):
        v = us_ref[...] + vd_ref[...] + _dot3(e_ref[...], w1cr) + b1r[...]
        v = jnp.maximum(v, 0.0)
        v = jnp.maximum(_dot3(v, w2r) + b2r[...], 0.0)
        v = _dot3(v, w3r) + b3r[...]
        o_ref[...] = e_ref[...] + _ln(v, gr[...], br[...])

    return pl.pallas_call(
        body,
        grid=(n // EBLK,),
        in_specs=[pl.BlockSpec((EBLK, D), _row)] * 3 + _full_specs(b1, w1c, w2, b2, w3, b3, g, b),
        out_specs=pl.BlockSpec((EBLK, D), _row),
        out_shape=jax.ShapeDtypeStruct((n, D), F32),
    )(us, vd, e, b1, w1c, w2, b2, w3, b3, g, b)


def _node_mlp(h, parts, w1h, w1a, b1, w2, b2, w3, b3, g, b):
    n = h.shape[0]
    blk = min(BLK, n)

    def body(h_ref, p_ref, w1hr, w1ar, b1r, w2r, b2r, w3r, b3r, gr, br, o_ref):
        agg = p_ref[0] + p_ref[1]
        v = _dot3(h_ref[...], w1hr) + _dot3(agg, w1ar) + b1r[...]
        v = jnp.maximum(v, 0.0)
        v = jnp.maximum(_dot3(v, w2r) + b2r[...], 0.0)
        v = _dot3(v, w3r) + b3r[...]
        o_ref[...] = h_ref[...] + _ln(v, gr[...], br[...])

    return pl.pallas_call(
        body,
        grid=(n // blk,),
        in_specs=[pl.BlockSpec((blk, D), _row), pl.BlockSpec((2, blk, D), lambda i: (0, i, 0))]
        + _full_specs(w1h, w1a, b1, w2, b2, w3, b3, g, b),
        out_specs=pl.BlockSpec((blk, D), _row),
        out_shape=jax.ShapeDtypeStruct((n, D), F32),
    )(h, parts, w1h, w1a, b1, w2, b2, w3, b3, g, b)


def _decoder(h, w1, b1, w2, b2, w3, b3):
    n = h.shape[0]
    blk = min(BLK, n)

    def body(h_ref, w1r, b1r, w2r, b2r, w3r, b3r, o_ref):
        v = jnp.maximum(_dot3(h_ref[...], w1r) + b1r[...], 0.0)
        v = jnp.maximum(_dot3(v, w2r) + b2r[...], 0.0)
        o_ref[...] = _dot3(v, w3r) + b3r[...]

    return pl.pallas_call(
        body,
        grid=(n // blk,),
        in_specs=[pl.BlockSpec((blk, D), _row)] + _full_specs(w1, b1, w2, b2, w3, b3),
        out_specs=pl.BlockSpec((blk, D), _row),
        out_shape=jax.ShapeDtypeStruct((n, D), F32),
    )(h, w1, b1, w2, b2, w3, b3)


def _sc_gather(u, v, src2d, dst2d):
    nb = src2d.shape[0]
    ep = nb * 128
    np_ = u.shape[0]
    rows = np_ // 16
    mesh = plsc.VectorSubcoreMesh(core_axis_name="c", subcore_axis_name="s")

    @functools.partial(
        pl.kernel,
        mesh=mesh,
        out_type=[
            jax.ShapeDtypeStruct((ep, D), F32),
            jax.ShapeDtypeStruct((ep, D), F32),
        ],
        scratch_types=[pltpu.VMEM_SHARED((np_, D), F32)],
    )
    def gk(u_hbm, v_hbm, s_hbm, d_hbm, us_hbm, vd_hbm, table_sh):
        cid = lax.axis_index("c")
        sid = lax.axis_index("s")
        sl = pl.ds(sid * rows, rows)

        @pl.when(cid == 0)
        def _():
            pltpu.sync_copy(u_hbm.at[sl], table_sh.at[sl])

        @pl.when(cid == 1)
        def _():
            pltpu.sync_copy(v_hbm.at[sl], table_sh.at[sl])

        plsc.subcore_barrier()

        def body(i_vmem, o_vmem):
            pltpu.sync_copy(table_sh.at[i_vmem.at[0]], o_vmem)

        pipe = functools.partial(
            pltpu.emit_pipeline,
            body,
            grid=(nb,),
            in_specs=[pl.BlockSpec((1, 128), _row)],
            out_specs=[pl.BlockSpec((128, D), _row)],
            core_axis_name="s",
            dimension_semantics=(pltpu.PARALLEL,),
        )

        @pl.when(cid == 0)
        def _():
            pipe()(s_hbm, us_hbm)

        @pl.when(cid == 1)
        def _():
            pipe()(d_hbm, vd_hbm)

    return gk(u, v, src2d, dst2d)


def _sc_gather_hbm(u, v, src2d, dst2d):
    nb = src2d.shape[0]
    ep = nb * 128
    mesh = plsc.VectorSubcoreMesh(core_axis_name="c", subcore_axis_name="s")

    @functools.partial(
        pl.kernel,
        mesh=mesh,
        out_type=[
            jax.ShapeDtypeStruct((ep, D), F32),
            jax.ShapeDtypeStruct((ep, D), F32),
        ],
    )
    def gk(u_hbm, v_hbm, s_hbm, d_hbm, us_hbm, vd_hbm):
        def body(s_vmem, d_vmem, us_vmem, vd_vmem):
            pltpu.sync_copy(u_hbm.at[s_vmem.at[0]], us_vmem)
            pltpu.sync_copy(v_hbm.at[d_vmem.at[0]], vd_vmem)

        pltpu.emit_pipeline(
            body,
            grid=(nb,),
            in_specs=[
                pl.BlockSpec((1, 128), _row),
                pl.BlockSpec((1, 128), _row),
            ],
            out_specs=[
                pl.BlockSpec((128, D), _row),
                pl.BlockSpec((128, D), _row),
            ],
            core_axis_name=("c", "s"),
            dimension_semantics=(pltpu.PARALLEL,),
        )(s_hbm, d_hbm, us_hbm, vd_hbm)

    return gk(u, v, src2d, dst2d)


def _sc_scatter(e_new, dst2d, zeros_blk):
    nb = dst2d.shape[0]
    rows = zeros_blk.shape[0]
    np_ = rows * 16
    mesh = plsc.VectorSubcoreMesh(core_axis_name="c", subcore_axis_name="s")

    @functools.partial(
        pl.kernel,
        mesh=mesh,
        out_type=jax.ShapeDtypeStruct((2, np_, D), F32),
        scratch_types=[pltpu.VMEM_SHARED((np_, D), F32)],
    )
    def sk(e_hbm, d_hbm, z_hbm, out_hbm, acc_shared):
        cid = lax.axis_index("c")
        sid = lax.axis_index("s")
        pltpu.sync_copy(z_hbm, acc_shared.at[pl.ds(sid * rows, rows)])
        plsc.subcore_barrier()

        def body(e_vmem, d_vmem):
            pltpu.sync_copy(e_vmem, acc_shared.at[d_vmem.at[0]], add=True)

        pltpu.emit_pipeline(
            body,
            grid=(nb,),
            in_specs=[
                pl.BlockSpec((128, D), _row),
                pl.BlockSpec((1, 128), _row),
            ],
            out_specs=[],
            core_axis_name=("c", "s"),
            dimension_semantics=(pltpu.PARALLEL,),
        )(e_hbm, d_hbm)

        plsc.subcore_barrier()
        pltpu.sync_copy(
            acc_shared.at[pl.ds(sid * rows, rows)],
            out_hbm.at[cid].at[pl.ds(sid * rows, rows)],
        )

    return sk(e_new, dst2d, zeros_blk)


def _sc_scatter2(e0, e1, d0, d1, zeros_blk):
    nb = d0.shape[0]
    rows = zeros_blk.shape[0]
    np_ = rows * 16
    mesh = plsc.VectorSubcoreMesh(core_axis_name="c", subcore_axis_name="s")

    @functools.partial(
        pl.kernel,
        mesh=mesh,
        out_type=jax.ShapeDtypeStruct((2, np_, D), F32),
        scratch_types=[pltpu.VMEM_SHARED((np_, D), F32)],
    )
    def sk(e0_hbm, e1_hbm, d0_hbm, d1_hbm, z_hbm, out_hbm, acc_shared):
        cid = lax.axis_index("c")
        sid = lax.axis_index("s")
        pltpu.sync_copy(z_hbm, acc_shared.at[pl.ds(sid * rows, rows)])
        plsc.subcore_barrier()

        def body(e_vmem, d_vmem):
            pltpu.sync_copy(e_vmem, acc_shared.at[d_vmem.at[0]], add=True)

        for e_hbm, d_hbm in ((e0_hbm, d0_hbm), (e1_hbm, d1_hbm)):
            pltpu.emit_pipeline(
                body,
                grid=(nb,),
                in_specs=[
                    pl.BlockSpec((128, D), _row),
                    pl.BlockSpec((1, 128), _row),
                ],
                out_specs=[],
                core_axis_name=("c", "s"),
                dimension_semantics=(pltpu.PARALLEL,),
            )(e_hbm, d_hbm)

        plsc.subcore_barrier()
        pltpu.sync_copy(
            acc_shared.at[pl.ds(sid * rows, rows)],
            out_hbm.at[cid].at[pl.ds(sid * rows, rows)],
        )

    return sk(e0, e1, d0, d1, zeros_blk)


def kernel(x, edge_index, edge_attr, params):
    n = x.shape[0]
    ne = edge_attr.shape[0]
    np_ = _round_up(n, 2048)
    ep = _round_up(ne, 8192)
    half = ep // 2
    hb = half // 128

    src = edge_index[0].astype(jnp.int32)
    dst = edge_index[1].astype(jnp.int32)
    pad_ids = (jnp.arange(ep - ne, dtype=jnp.int32) % (np_ - n)) + n
    src2d = jnp.concatenate([src, pad_ids]).reshape(ep // 128, 128)
    dst2d = jnp.concatenate([dst, pad_ids]).reshape(ep // 128, 128)
    schunks = (src2d[:hb], src2d[hb:])
    dchunks = (dst2d[:hb], dst2d[hb:])

    x_pad = jnp.pad(x, ((0, np_ - n), (0, 0)))
    ea_pad = jnp.pad(edge_attr, ((0, ep - ne), (0, 0)))
    ea_chunks = (ea_pad[:half], ea_pad[half:])
    zeros_blk = jnp.zeros((np_ // 16, D), F32)

    def unpack(p, split_first=True):
        lin = p["lin"]
        out = []
        for i, l in enumerate(lin):
            out.append(_split_w(l["w"]) if (split_first or i > 0) else l["w"])
            out.append(l["b"].reshape(1, -1))
        if "ln" in p:
            out.append(p["ln"]["g"].reshape(1, -1))
            out.append(p["ln"]["b"].reshape(1, -1))
        return out

    h = _node_enc(x_pad, *unpack(params["node_enc"]))
    enc_w = unpack(params["edge_enc"], split_first=False)
    e = [_edge_enc(ea_c, *enc_w) for ea_c in ea_chunks]

    for blk in params["blocks"]:
        w1 = blk["edge_mlp"]["lin"][0]["w"]
        ew = unpack(blk["edge_mlp"])[1:]
        w1cp = _split_w(w1[2 * D :])
        u, v = _uv(h, _split_w(w1[:D]), _split_w(w1[D : 2 * D]))
        gath = [_sc_gather(u, v, schunks[c], dchunks[c]) for c in range(2)]
        e = [
            _edge_mlp(gath[c][0], gath[c][1], e[c], ew[0], w1cp, *ew[1:])
            for c in range(2)
        ]
        parts = _sc_scatter2(e[0], e[1], dchunks[0], dchunks[1], zeros_blk)
        w1n = blk["node_mlp"]["lin"][0]["w"]
        nw = unpack(blk["node_mlp"])[1:]
        h = _node_mlp(h, parts, _split_w(w1n[:D]), _split_w(w1n[D:]), *nw)

    dw = unpack(params["node_dec"])
    w3 = params["node_dec"]["lin"][2]["w"]
    out_dim = w3.shape[1]
    w3p = _split_w(jnp.pad(w3, ((0, 0), (0, D - out_dim))))
    b3p = jnp.pad(dw[5], ((0, 0), (0, D - out_dim)))
    out = _decoder(h, dw[0], dw[1], dw[2], dw[3], w3p, b3p)
    return out[:n, :out_dim]
